# baseline (speedup 1.0000x reference)
import jax
import jax.numpy as jnp
from jax import lax
from jax.experimental import pallas as pl
from jax.experimental.pallas import tpu as pltpu
from jax.experimental.pallas import tpu_sc as plsc

_N_ROWS = 100000
_DIM = 128
_SCALE = 1.0 / (128.0 ** 0.5)
_C = 80
_NW = 32
_PW = 40
_WSPAN = _PW * _C
_WLAST = _N_ROWS - _WSPAN
_NBUF = 10
_SKEW = _NBUF // 2
_TROWS = _DIM // 16


def _gather_body(table_hbm, idx_hbm, out_hbm, tbl_sh, idx_v, rows_v, tstage,
                 *sems):
    sg = sems[:_NBUF]
    so = sems[_NBUF:2 * _NBUF]
    si = sems[2 * _NBUF]
    s = lax.axis_index("s")
    w = s * 2 + lax.axis_index("c")
    base = jnp.minimum(w * _WSPAN, _WLAST)
    base = pl.multiple_of(base, 8)

    pltpu.async_copy(idx_hbm.at[pl.ds(base, _WSPAN)], idx_v, si)

    trow = pl.multiple_of(s * _TROWS, 8)
    pltpu.sync_copy(table_hbm.at[pl.ds(trow, _TROWS)], tstage)
    for r in range(_TROWS):
        for c in range(_DIM // 16):
            tstage[r, pl.ds(16 * c, 16)] = tstage[r, pl.ds(16 * c, 16)] * _SCALE
    pltpu.sync_copy(tstage, tbl_sh.at[pl.ds(trow, _TROWS)])

    pltpu.make_async_copy(idx_hbm.at[pl.ds(base, _WSPAN)], idx_v, si).wait()
    plsc.subcore_barrier()

    def gather_pair(i, b):
        ii = jnp.minimum(i, _PW - 1)
        sl = idx_v.at[pl.ds(pl.multiple_of(ii * _C, 8), _C)]
        return tbl_sh.at[sl], rows_v.at[b]

    def start_gather(i, b):
        src, dst = gather_pair(i, b)
        pltpu.async_copy(src, dst, sg[b])

    def wait_gather(i, b):
        src, dst = gather_pair(i, b)
        pltpu.make_async_copy(src, dst, sg[b]).wait()

    def wb_pair(i, b):
        ii = jnp.minimum(i, _PW - 1)
        off = pl.multiple_of(base + ii * _C, 8)
        return rows_v.at[b], out_hbm.at[pl.ds(off, _C)]

    def start_wb(i, b):
        src, dst = wb_pair(i, b)
        pltpu.async_copy(src, dst, so[b])

    def wait_wb(i, b):
        src, dst = wb_pair(i, b)
        pltpu.make_async_copy(src, dst, so[b]).wait()

    def step(i, b, first):
        bn = (b + _SKEW) % _NBUF
        wait_gather(i, b)
        start_wb(i, b)
        if not first:
            wait_wb(i - _SKEW, bn)
        start_gather(i + _SKEW, bn)

    for b in range(_SKEW):
        start_gather(jnp.int32(b), b)
    for i in range(_SKEW):
        step(jnp.int32(i), i, True)

    def wave(j, carry):
        i0 = _SKEW + j * _NBUF
        for u in range(_NBUF):
            step(i0 + u, (_SKEW + u) % _NBUF, False)
        return carry

    lax.fori_loop(0, (_PW - 2 * _SKEW) // _NBUF, wave, 0)

    for u in range(_SKEW):
        i = _PW - _SKEW + u
        step(jnp.int32(i), i % _NBUF, False)

    for b in range(_SKEW):
        wait_gather(jnp.int32(_PW - 1), b)
    for u in range(_SKEW):
        i = _PW - _SKEW + u
        wait_wb(jnp.int32(i), i % _NBUF)


def kernel(node_specie, embeddings):
    idx = node_specie.astype(jnp.int32)
    mesh = plsc.VectorSubcoreMesh(core_axis_name="c", subcore_axis_name="s")
    f = pl.kernel(
        _gather_body,
        mesh=mesh,
        out_type=jax.ShapeDtypeStruct((_N_ROWS, _DIM), jnp.float32),
        scratch_types=[
            pltpu.VMEM_SHARED((_DIM, _DIM), jnp.float32),
            pltpu.VMEM((_WSPAN,), jnp.int32),
            pltpu.VMEM((_NBUF, _C, _DIM), jnp.float32),
            pltpu.VMEM((_TROWS, _DIM), jnp.float32),
        ] + [pltpu.SemaphoreType.DMA] * (2 * _NBUF + 1),
    )
    return f(embeddings, idx)


# speedup vs baseline: 1.0158x; 1.0158x over previous
import jax
import jax.numpy as jnp
from jax import lax
from jax.experimental import pallas as pl
from jax.experimental.pallas import tpu as pltpu
from jax.experimental.pallas import tpu_sc as plsc

_N_ROWS = 100000
_DIM = 128
_SCALE = 1.0 / (128.0 ** 0.5)
_C = 80
_NW = 32
_PW = 40
_WSPAN = _PW * _C
_WLAST = _N_ROWS - _WSPAN
_NBUF = 8
_SKEW = _NBUF // 2
_TROWS = _DIM // 16


def _gather_body(table_hbm, idx_hbm, out_hbm, tbl_sh, idx_v, rows_v, tstage,
                 *sems):
    sg = sems[:_NBUF]
    so = sems[_NBUF:2 * _NBUF]
    si = sems[2 * _NBUF]
    s = lax.axis_index("s")
    w = s * 2 + lax.axis_index("c")
    base = jnp.minimum(w * _WSPAN, _WLAST)
    base = pl.multiple_of(base, 8)

    pltpu.async_copy(idx_hbm.at[pl.ds(base, _WSPAN)], idx_v, si)

    trow = pl.multiple_of(s * _TROWS, 8)
    pltpu.sync_copy(table_hbm.at[pl.ds(trow, _TROWS)], tstage)
    for r in range(_TROWS):
        for c in range(_DIM // 16):
            tstage[r, pl.ds(16 * c, 16)] = tstage[r, pl.ds(16 * c, 16)] * _SCALE
    pltpu.sync_copy(tstage, tbl_sh.at[pl.ds(trow, _TROWS)])

    pltpu.make_async_copy(idx_hbm.at[pl.ds(base, _WSPAN)], idx_v, si).wait()
    plsc.subcore_barrier()

    def gather_pair(i, b):
        ii = jnp.minimum(i, _PW - 1)
        sl = idx_v.at[pl.ds(pl.multiple_of(ii * _C, 8), _C)]
        return tbl_sh.at[sl], rows_v.at[b]

    def start_gather(i, b):
        src, dst = gather_pair(i, b)
        pltpu.async_copy(src, dst, sg[b])

    def wait_gather(i, b):
        src, dst = gather_pair(i, b)
        pltpu.make_async_copy(src, dst, sg[b]).wait()

    def wb_pair(i, b):
        ii = jnp.minimum(i, _PW - 1)
        off = pl.multiple_of(base + ii * _C, 8)
        return rows_v.at[b], out_hbm.at[pl.ds(off, _C)]

    def start_wb(i, b):
        src, dst = wb_pair(i, b)
        pltpu.async_copy(src, dst, so[b])

    def wait_wb(i, b):
        src, dst = wb_pair(i, b)
        pltpu.make_async_copy(src, dst, so[b]).wait()

    def step(i, b, first):
        bn = (b + _SKEW) % _NBUF
        if not first:
            wait_wb(i - _SKEW, bn)
        start_gather(i + _SKEW, bn)
        wait_gather(i, b)
        start_wb(i, b)

    for b in range(_SKEW):
        start_gather(jnp.int32(b), b)
    for i in range(_SKEW):
        step(jnp.int32(i), i, True)

    def wave(j, carry):
        i0 = _SKEW + j * _NBUF
        for u in range(_NBUF):
            step(i0 + u, (_SKEW + u) % _NBUF, False)
        return carry

    lax.fori_loop(0, (_PW - 2 * _SKEW) // _NBUF, wave, 0)

    for u in range(_SKEW):
        i = _PW - _SKEW + u
        step(jnp.int32(i), i % _NBUF, False)

    for b in range(_SKEW):
        wait_gather(jnp.int32(_PW - 1), b)
    for u in range(_SKEW):
        i = _PW - _SKEW + u
        wait_wb(jnp.int32(i), i % _NBUF)


def kernel(node_specie, embeddings):
    idx = node_specie.astype(jnp.int32)
    mesh = plsc.VectorSubcoreMesh(core_axis_name="c", subcore_axis_name="s")
    f = pl.kernel(
        _gather_body,
        mesh=mesh,
        out_type=jax.ShapeDtypeStruct((_N_ROWS, _DIM), jnp.float32),
        scratch_types=[
            pltpu.VMEM_SHARED((_DIM, _DIM), jnp.float32),
            pltpu.VMEM((_WSPAN,), jnp.int32),
            pltpu.VMEM((_NBUF, _C, _DIM), jnp.float32),
            pltpu.VMEM((_TROWS, _DIM), jnp.float32),
        ] + [pltpu.SemaphoreType.DMA] * (2 * _NBUF + 1),
    )
    return f(embeddings, idx)
